# Initial kernel scaffold; baseline (speedup 1.0000x reference)
import jax
import jax.numpy as jnp
from jax.experimental import pallas as pl


def kernel(embedding_support, embedding_query, embedding_global_support, embedding_global_query, memory_keys, memory_values):
    raise NotImplementedError("write your pallas kernel here")


# speedup vs baseline: 26.5672x; 26.5672x over previous
import jax
import jax.numpy as jnp
from jax.experimental import pallas as pl

_T = 4
_N = 100
_R = _T * _N
_D = 512
_M = 512
_THRESH = 0.5
_QK = 0.5
_MARGIN = 0.1


def _l2rows(x):
    n = jnp.sqrt(jnp.sum(x * x, axis=-1, keepdims=True))
    return x / jnp.maximum(n, 1e-12)


def _dot_nt(a, b):
    return jax.lax.dot_general(
        a, b, (((1,), (1,)), ((), ())),
        preferred_element_type=jnp.float32,
        precision=jax.lax.Precision.HIGHEST)


def _dot_tn(a, b):
    return jax.lax.dot_general(
        a, b, (((0,), (0,)), ((), ())),
        preferred_element_type=jnp.float32,
        precision=jax.lax.Precision.HIGHEST)


def _dot_nn(a, b):
    return jax.lax.dot_general(
        a, b, (((1,), (0,)), ((), ())),
        preferred_element_type=jnp.float32,
        precision=jax.lax.Precision.HIGHEST)


def _first_argmax_onehot(sim, iota):
    mx = jnp.max(sim, axis=1, keepdims=True)
    idx = jnp.min(jnp.where(sim == mx, iota, _M), axis=1, keepdims=True)
    return (iota == idx).astype(jnp.float32)


def _first_argmin_onehot(sim, iota):
    mn = jnp.min(sim, axis=1, keepdims=True)
    idx = jnp.min(jnp.where(sim == mn, iota, _M), axis=1, keepdims=True)
    return (iota == idx).astype(jnp.float32)


def _fused(emb_ref, glo_ref, mk_ref, mv_ref,
           ne_ref, eg_ref, lk_ref, lv_ref):
    ne = _l2rows(emb_ref[...])
    ng = _l2rows(glo_ref[...])
    mk = mk_ref[...]
    mv = mv_ref[...]

    mk_n = _l2rows(mk)
    mv_n = _l2rows(mv)

    iota = jax.lax.broadcasted_iota(jnp.int32, (_R, _M), 1)

    sim_kv = _dot_nt(ne, mk_n)
    score = jnp.where(sim_kv >= _THRESH, sim_kv, 0.0)
    mvu = _l2rows(_QK * mv + ((1.0 - _QK) / _R) * _dot_tn(score, ng))

    sim_vk = _dot_nt(ng, mv_n)
    oh_vk = _first_argmax_onehot(sim_vk, iota)
    mku = _l2rows(_QK * mk + ((1.0 - _QK) / _R) * _dot_tn(oh_vk, ne))

    sim_kv2 = _dot_nt(ne, mku)
    sim_vk2 = _dot_nt(ng, mvu)

    eg = _l2rows(ng + _dot_nn(sim_kv2, mvu))

    ne_ref[...] = ne
    eg_ref[...] = eg

    ng_sq = jnp.sum(ng * ng, axis=1, keepdims=True)
    ne_sq = jnp.sum(ne * ne, axis=1, keepdims=True)
    mvu_sq = jnp.sum(mvu * mvu, axis=1).reshape(1, _M)
    mku_sq = jnp.sum(mku * mku, axis=1).reshape(1, _M)

    oh_v = _first_argmax_onehot(sim_kv2, iota)
    sel_sq = jnp.sum(oh_v * mvu_sq, axis=1, keepdims=True)
    sel_dot = jnp.sum(oh_v * sim_vk2, axis=1, keepdims=True)
    loss_v_col = sel_sq - 2.0 * sel_dot + ng_sq
    lv_ref[...] = jnp.sum(loss_v_col, axis=0, keepdims=True) / _R

    oh_kmax = _first_argmax_onehot(sim_vk2, iota)
    oh_kmin = _first_argmin_onehot(sim_vk2, iota)
    lmax_col = (jnp.sum(oh_kmax * mku_sq, axis=1, keepdims=True)
                - 2.0 * jnp.sum(oh_kmax * sim_kv2, axis=1, keepdims=True)
                + ne_sq)
    lmin_col = (jnp.sum(oh_kmin * mku_sq, axis=1, keepdims=True)
                - 2.0 * jnp.sum(oh_kmin * sim_kv2, axis=1, keepdims=True)
                + ne_sq)
    diff = jnp.sum(lmax_col - lmin_col, axis=0, keepdims=True) / _R
    lk_ref[...] = jnp.maximum(diff + _MARGIN, 0.0)


def kernel(embedding_support, embedding_query,
           embedding_global_support, embedding_global_query,
           memory_keys, memory_values):
    emb = jnp.concatenate([embedding_support, embedding_query], axis=1)
    glo = jnp.concatenate([embedding_global_support, embedding_global_query],
                          axis=1)
    emb2 = emb.reshape(_R, _D)
    glo2 = glo.reshape(_R, _D)

    ne, eg, lk, lv = pl.pallas_call(
        _fused,
        out_shape=[
            jax.ShapeDtypeStruct((_R, _D), jnp.float32),
            jax.ShapeDtypeStruct((_R, _D), jnp.float32),
            jax.ShapeDtypeStruct((1, 1), jnp.float32),
            jax.ShapeDtypeStruct((1, 1), jnp.float32),
        ],
    )(emb2, glo2, memory_keys, memory_values)

    out = jnp.concatenate([ne.reshape(_T, _N, _D), eg.reshape(_T, _N, _D)],
                          axis=-1)
    return out, lk.reshape(()), lv.reshape(())


# speedup vs baseline: 40.1748x; 1.5122x over previous
import jax
import jax.numpy as jnp
from jax.experimental import pallas as pl

_T = 4
_N = 100
_R = _T * _N
_D = 512
_M = 512
_THRESH = 0.5
_QK = 0.5
_MARGIN = 0.1


def _l2rows(x):
    n = jnp.sqrt(jnp.sum(x * x, axis=-1, keepdims=True))
    return x / jnp.maximum(n, 1e-12)


def _dot_nt(a, b):
    return jax.lax.dot_general(
        a, b, (((1,), (1,)), ((), ())),
        preferred_element_type=jnp.float32,
        precision=jax.lax.Precision.DEFAULT)


def _dot_tn(a, b):
    return jax.lax.dot_general(
        a, b, (((0,), (0,)), ((), ())),
        preferred_element_type=jnp.float32,
        precision=jax.lax.Precision.DEFAULT)


def _dot_nn(a, b):
    return jax.lax.dot_general(
        a, b, (((1,), (0,)), ((), ())),
        preferred_element_type=jnp.float32,
        precision=jax.lax.Precision.DEFAULT)


def _first_argmax_onehot(sim, iota):
    mx = jnp.max(sim, axis=1, keepdims=True)
    idx = jnp.min(jnp.where(sim == mx, iota, _M), axis=1, keepdims=True)
    return (iota == idx).astype(jnp.float32)


def _first_argmin_onehot(sim, iota):
    mn = jnp.min(sim, axis=1, keepdims=True)
    idx = jnp.min(jnp.where(sim == mn, iota, _M), axis=1, keepdims=True)
    return (iota == idx).astype(jnp.float32)


def _fused(emb_ref, glo_ref, mk_ref, mv_ref,
           ne_ref, eg_ref, lk_ref, lv_ref):
    ne = _l2rows(emb_ref[...])
    ng = _l2rows(glo_ref[...])
    mk = mk_ref[...]
    mv = mv_ref[...]

    mk_n = _l2rows(mk)
    mv_n = _l2rows(mv)

    iota = jax.lax.broadcasted_iota(jnp.int32, (_R, _M), 1)

    sim_kv = _dot_nt(ne, mk_n)
    score = jnp.where(sim_kv >= _THRESH, sim_kv, 0.0)
    mvu = _l2rows(_QK * mv + ((1.0 - _QK) / _R) * _dot_tn(score, ng))

    sim_vk = _dot_nt(ng, mv_n)
    oh_vk = _first_argmax_onehot(sim_vk, iota)
    mku = _l2rows(_QK * mk + ((1.0 - _QK) / _R) * _dot_tn(oh_vk, ne))

    sim_kv2 = _dot_nt(ne, mku)
    sim_vk2 = _dot_nt(ng, mvu)

    eg = _l2rows(ng + _dot_nn(sim_kv2, mvu))

    ne_ref[...] = ne
    eg_ref[...] = eg

    ng_sq = jnp.sum(ng * ng, axis=1, keepdims=True)
    ne_sq = jnp.sum(ne * ne, axis=1, keepdims=True)
    mvu_sq = jnp.sum(mvu * mvu, axis=1).reshape(1, _M)
    mku_sq = jnp.sum(mku * mku, axis=1).reshape(1, _M)

    oh_v = _first_argmax_onehot(sim_kv2, iota)
    sel_sq = jnp.sum(oh_v * mvu_sq, axis=1, keepdims=True)
    sel_dot = jnp.sum(oh_v * sim_vk2, axis=1, keepdims=True)
    loss_v_col = sel_sq - 2.0 * sel_dot + ng_sq
    lv_ref[...] = jnp.sum(loss_v_col, axis=0, keepdims=True) / _R

    oh_kmax = _first_argmax_onehot(sim_vk2, iota)
    oh_kmin = _first_argmin_onehot(sim_vk2, iota)
    lmax_col = (jnp.sum(oh_kmax * mku_sq, axis=1, keepdims=True)
                - 2.0 * jnp.sum(oh_kmax * sim_kv2, axis=1, keepdims=True)
                + ne_sq)
    lmin_col = (jnp.sum(oh_kmin * mku_sq, axis=1, keepdims=True)
                - 2.0 * jnp.sum(oh_kmin * sim_kv2, axis=1, keepdims=True)
                + ne_sq)
    diff = jnp.sum(lmax_col - lmin_col, axis=0, keepdims=True) / _R
    lk_ref[...] = jnp.maximum(diff + _MARGIN, 0.0)


def kernel(embedding_support, embedding_query,
           embedding_global_support, embedding_global_query,
           memory_keys, memory_values):
    emb = jnp.concatenate([embedding_support, embedding_query], axis=1)
    glo = jnp.concatenate([embedding_global_support, embedding_global_query],
                          axis=1)
    emb2 = emb.reshape(_R, _D)
    glo2 = glo.reshape(_R, _D)

    ne, eg, lk, lv = pl.pallas_call(
        _fused,
        out_shape=[
            jax.ShapeDtypeStruct((_R, _D), jnp.float32),
            jax.ShapeDtypeStruct((_R, _D), jnp.float32),
            jax.ShapeDtypeStruct((1, 1), jnp.float32),
            jax.ShapeDtypeStruct((1, 1), jnp.float32),
        ],
    )(emb2, glo2, memory_keys, memory_values)

    out = jnp.concatenate([ne.reshape(_T, _N, _D), eg.reshape(_T, _N, _D)],
                          axis=-1)
    return out, lk.reshape(()), lv.reshape(())
